# baseline (speedup 1.0000x reference)
import functools

import jax
import jax.numpy as jnp
from jax import lax
from jax.experimental import pallas as pl
from jax.experimental.pallas import tpu as pltpu
from jax.experimental.pallas import tpu_sc as plsc

BATCH = 4096
SEQ = 50
EMB = 300
EMBP = 384
VOCAB = 100000
N = BATCH * SEQ

NC = 2
NS = 16
NW = NC * NS
PER_W = N // NW
CHUNK = 128
NCH = PER_W // CHUNK

_mesh = plsc.VectorSubcoreMesh(
    core_axis_name="c", subcore_axis_name="s", num_cores=NC, num_subcores=NS
)


@functools.partial(
    pl.kernel,
    out_type=jax.ShapeDtypeStruct((N, EMBP), jnp.float32),
    mesh=_mesh,
    scratch_types=[
        pltpu.VMEM((PER_W,), jnp.int32),
        pltpu.VMEM((CHUNK, EMBP), jnp.float32),
        pltpu.VMEM((CHUNK, EMBP), jnp.float32),
        pltpu.SemaphoreType.DMA,
        pltpu.SemaphoreType.DMA,
    ],
    compiler_params=pltpu.CompilerParams(use_tc_tiling_on_sc=True),
)
def _emb_lookup(idx_hbm, table_hbm, tail_hbm, out_hbm, idx_v, buf0, buf1, sem0, sem1):
    wid = lax.axis_index("s") * NC + lax.axis_index("c")
    base = wid * PER_W
    pltpu.sync_copy(idx_hbm.at[pl.ds(base, PER_W)], idx_v)

    bufs = (buf0, buf1)
    sems = (sem0, sem1)

    def start_gathers(g, b):
        idx = idx_v.at[pl.ds(g * CHUNK, CHUNK)]
        return (
            pltpu.async_copy(table_hbm.at[idx, pl.ds(0, 128)],
                             bufs[b].at[:, pl.ds(0, 128)], sems[b]),
            pltpu.async_copy(table_hbm.at[idx, pl.ds(128, 128)],
                             bufs[b].at[:, pl.ds(128, 128)], sems[b]),
            pltpu.async_copy(tail_hbm.at[idx],
                             bufs[b].at[:, pl.ds(256, 128)], sems[b]),
        )

    def store(g, b):
        pltpu.sync_copy(bufs[b], out_hbm.at[pl.ds(base + g * CHUNK, CHUNK)])

    @pl.loop(0, NCH, step=2)
    def _pair(g):
        cps0 = start_gathers(g, 0)
        cps1 = start_gathers(g + 1, 1)
        for cp in cps0:
            cp.wait()
        store(g, 0)
        for cp in cps1:
            cp.wait()
        store(g + 1, 1)


def kernel(indices, table):
    idx_flat = indices.reshape(N)
    tail = jnp.pad(table[:, 256:], ((0, 0), (0, EMBP - EMB)))
    out = _emb_lookup(idx_flat, table, tail)
    return out[:, :EMB].reshape(BATCH, SEQ, EMB)


# speedup vs baseline: 1.1254x; 1.1254x over previous
import functools

import jax
import jax.numpy as jnp
from jax import lax
from jax.experimental import pallas as pl
from jax.experimental.pallas import tpu as pltpu
from jax.experimental.pallas import tpu_sc as plsc

BATCH = 4096
SEQ = 50
SEQP = 56
EMB = 300
EMBP = 384
VOCAB = 100000
NP = BATCH * SEQP

NC = 2
NS = 16
NW = NC * NS
PER_W = NP // NW
CHUNK = 128
NCH = PER_W // CHUNK

_mesh = plsc.VectorSubcoreMesh(
    core_axis_name="c", subcore_axis_name="s", num_cores=NC, num_subcores=NS
)


@functools.partial(
    pl.kernel,
    out_type=jax.ShapeDtypeStruct((NP, EMBP), jnp.float32),
    mesh=_mesh,
    scratch_types=[
        pltpu.VMEM((PER_W,), jnp.int32),
        pltpu.VMEM((CHUNK, EMBP), jnp.float32),
        pltpu.VMEM((CHUNK, EMBP), jnp.float32),
        pltpu.SemaphoreType.DMA,
        pltpu.SemaphoreType.DMA,
    ],
    compiler_params=pltpu.CompilerParams(use_tc_tiling_on_sc=True),
)
def _emb_lookup(idx_hbm, table_hbm, tail_hbm, out_hbm, idx_v, buf0, buf1, sem0, sem1):
    wid = lax.axis_index("s") * NC + lax.axis_index("c")
    base = wid * PER_W
    pltpu.sync_copy(idx_hbm.at[pl.ds(base, PER_W)], idx_v)

    bufs = (buf0, buf1)
    sems = (sem0, sem1)

    def start_gathers(g, b):
        idx = idx_v.at[pl.ds(g * CHUNK, CHUNK)]
        return (
            pltpu.async_copy(table_hbm.at[idx, pl.ds(0, 128)],
                             bufs[b].at[:, pl.ds(0, 128)], sems[b]),
            pltpu.async_copy(table_hbm.at[idx, pl.ds(128, 128)],
                             bufs[b].at[:, pl.ds(128, 128)], sems[b]),
            pltpu.async_copy(tail_hbm.at[idx],
                             bufs[b].at[:, pl.ds(256, 128)], sems[b]),
        )

    def store(g, b):
        pltpu.sync_copy(bufs[b], out_hbm.at[pl.ds(base + g * CHUNK, CHUNK)])

    @pl.loop(0, NCH, step=2)
    def _pair(g):
        cps0 = start_gathers(g, 0)
        cps1 = start_gathers(g + 1, 1)
        for cp in cps0:
            cp.wait()
        store(g, 0)
        for cp in cps1:
            cp.wait()
        store(g + 1, 1)


def kernel(indices, table):
    dummy = (jnp.arange(BATCH * (SEQP - SEQ), dtype=jnp.int32) % VOCAB).reshape(
        BATCH, SEQP - SEQ
    )
    idx_pad = jnp.concatenate([indices, dummy], axis=1).reshape(NP)
    tail = jnp.pad(table[:, 256:], ((0, 0), (0, EMBP - EMB)))
    out = _emb_lookup(idx_pad, table, tail)
    one = jnp.float32(1.0) + jnp.float32(0.0) * table[0, 0]
    return out.reshape(BATCH, SEQP, EMBP)[:, :SEQ, :EMB] * one


# speedup vs baseline: 1.3843x; 1.2301x over previous
import functools

import jax
import jax.numpy as jnp
from jax import lax
from jax.experimental import pallas as pl
from jax.experimental.pallas import tpu as pltpu
from jax.experimental.pallas import tpu_sc as plsc

BATCH = 4096
SEQ = 50
SEQP = 56
EMB = 300
EMBP = 384
VOCAB = 100000
NP = BATCH * SEQP

NC = 2
NS = 16
NW = NC * NS
PER_W = NP // NW
CHUNK = 128
NCH = PER_W // CHUNK

_mesh = plsc.VectorSubcoreMesh(
    core_axis_name="c", subcore_axis_name="s", num_cores=NC, num_subcores=NS
)


@functools.partial(
    pl.kernel,
    out_type=jax.ShapeDtypeStruct((NP, EMBP), jnp.float32),
    mesh=_mesh,
    scratch_types=[
        pltpu.VMEM((PER_W,), jnp.int32),
        pltpu.VMEM((CHUNK, EMBP), jnp.float32),
        pltpu.VMEM((CHUNK, EMBP), jnp.float32),
        pltpu.SemaphoreType.DMA,
        pltpu.SemaphoreType.DMA,
        pltpu.SemaphoreType.DMA,
        pltpu.SemaphoreType.DMA,
    ],
    compiler_params=pltpu.CompilerParams(use_tc_tiling_on_sc=True),
)
def _emb_lookup(idx_hbm, table_hbm, tail_hbm, out_hbm, idx_v, buf0, buf1,
                gsem0, gsem1, ssem0, ssem1):
    wid = lax.axis_index("s") * NC + lax.axis_index("c")
    base = wid * PER_W
    pltpu.sync_copy(idx_hbm.at[pl.ds(base, PER_W)], idx_v)

    bufs = (buf0, buf1)
    gsems = (gsem0, gsem1)
    ssems = (ssem0, ssem1)

    def start_gathers(g, b):
        idx = idx_v.at[pl.ds(g * CHUNK, CHUNK)]
        pltpu.async_copy(table_hbm.at[idx, pl.ds(0, 128)],
                         bufs[b].at[:, pl.ds(0, 128)], gsems[b])
        pltpu.async_copy(table_hbm.at[idx, pl.ds(128, 128)],
                         bufs[b].at[:, pl.ds(128, 128)], gsems[b])
        pltpu.async_copy(tail_hbm.at[idx],
                         bufs[b].at[:, pl.ds(256, 128)], gsems[b])

    def wait_gathers(b):
        pltpu.make_async_copy(out_hbm.at[pl.ds(0, CHUNK)], bufs[b],
                              gsems[b]).wait()

    def start_store(g, b):
        pltpu.async_copy(bufs[b], out_hbm.at[pl.ds(base + g * CHUNK, CHUNK)],
                         ssems[b])

    def wait_store(b):
        pltpu.make_async_copy(bufs[b], out_hbm.at[pl.ds(0, CHUNK)],
                              ssems[b]).wait()

    start_gathers(0, 0)
    start_gathers(1, 1)

    @pl.loop(0, NCH, step=2)
    def _pair(g):
        wait_gathers(0)
        start_store(g, 0)
        wait_gathers(1)
        start_store(g + 1, 1)

        @pl.when(g + 2 < NCH)
        def _():
            wait_store(0)
            start_gathers(g + 2, 0)

        @pl.when(g + 3 < NCH)
        def _():
            wait_store(1)
            start_gathers(g + 3, 1)

    wait_store(0)
    wait_store(1)


def kernel(indices, table):
    dummy = (jnp.arange(BATCH * (SEQP - SEQ), dtype=jnp.int32) % VOCAB).reshape(
        BATCH, SEQP - SEQ
    )
    idx_pad = jnp.concatenate([indices, dummy], axis=1).reshape(NP)
    tail = jnp.pad(table[:, 256:], ((0, 0), (0, EMBP - EMB)))
    out = _emb_lookup(idx_pad, table, tail)
    return out.reshape(BATCH, SEQP, EMBP)[:, :SEQ, :EMB]


# speedup vs baseline: 1.3862x; 1.0014x over previous
import functools

import jax
import jax.numpy as jnp
from jax import lax
from jax.experimental import pallas as pl
from jax.experimental.pallas import tpu as pltpu
from jax.experimental.pallas import tpu_sc as plsc

BATCH = 4096
SEQ = 50
SEQP = 56
EMB = 300
EMBP = 384
VOCAB = 100000
NP = BATCH * SEQP

NC = 2
NS = 16
NW = NC * NS
PER_W = NP // NW
CHUNK = 128
NCH = PER_W // CHUNK

_mesh = plsc.VectorSubcoreMesh(
    core_axis_name="c", subcore_axis_name="s", num_cores=NC, num_subcores=NS
)


@functools.partial(
    pl.kernel,
    out_type=jax.ShapeDtypeStruct((NP, EMBP), jnp.float32),
    mesh=_mesh,
    scratch_types=[
        pltpu.VMEM((PER_W,), jnp.int32),
        pltpu.VMEM((CHUNK, EMBP), jnp.float32),
        pltpu.VMEM((CHUNK, EMBP), jnp.float32),
        pltpu.SemaphoreType.DMA,
        pltpu.SemaphoreType.DMA,
        pltpu.SemaphoreType.DMA,
        pltpu.SemaphoreType.DMA,
    ],
    compiler_params=pltpu.CompilerParams(use_tc_tiling_on_sc=True),
)
def _emb_lookup(idx_hbm, table_hbm, tail_hbm, out_hbm, idx_v, buf0, buf1,
                gsem0, gsem1, ssem0, ssem1):
    wid = lax.axis_index("s") * NC + lax.axis_index("c")
    base = wid * PER_W
    pltpu.sync_copy(idx_hbm.at[pl.ds(base, PER_W)], idx_v)

    bufs = (buf0, buf1)
    gsems = (gsem0, gsem1)
    ssems = (ssem0, ssem1)

    def start_gathers(g, b):
        idx = idx_v.at[pl.ds(g * CHUNK, CHUNK)]
        pltpu.async_copy(table_hbm.at[idx, pl.ds(0, 128)],
                         bufs[b].at[:, pl.ds(0, 128)], gsems[b])
        pltpu.async_copy(table_hbm.at[idx, pl.ds(128, 128)],
                         bufs[b].at[:, pl.ds(128, 128)], gsems[b])
        pltpu.async_copy(tail_hbm.at[idx],
                         bufs[b].at[:, pl.ds(256, 128)], gsems[b])

    def wait_gathers(b):
        pltpu.make_async_copy(out_hbm.at[pl.ds(0, CHUNK)], bufs[b],
                              gsems[b]).wait()

    def start_store(g, b):
        pltpu.async_copy(bufs[b], out_hbm.at[pl.ds(base + g * CHUNK, CHUNK)],
                         ssems[b])

    def wait_store(b):
        pltpu.make_async_copy(bufs[b], out_hbm.at[pl.ds(0, CHUNK)],
                              ssems[b]).wait()

    start_gathers(0, 0)
    start_gathers(1, 1)

    @pl.loop(0, NCH, step=2)
    def _pair(g):
        wait_gathers(0)
        start_store(g, 0)
        wait_gathers(1)
        start_store(g + 1, 1)

        @pl.when(g + 2 < NCH)
        def _():
            wait_store(0)
            start_gathers(g + 2, 0)

        @pl.when(g + 3 < NCH)
        def _():
            wait_store(1)
            start_gathers(g + 3, 1)

    wait_store(0)
    wait_store(1)


def kernel(indices, table):
    dummy = (jnp.arange(BATCH * (SEQP - SEQ), dtype=jnp.int32) % VOCAB).reshape(
        BATCH, SEQP - SEQ
    )
    idx_pad = jnp.concatenate([indices, dummy], axis=1).reshape(NP)
    tail = jnp.concatenate(
        [table[:, 256:], jnp.zeros((VOCAB, EMBP - EMB), jnp.float32)], axis=1
    )
    out = _emb_lookup(idx_pad, table, tail)
    return out.reshape(BATCH, SEQP, EMBP)[:, :SEQ, :EMB]


# speedup vs baseline: 1.3872x; 1.0007x over previous
import functools

import jax
import jax.numpy as jnp
from jax import lax
from jax.experimental import pallas as pl
from jax.experimental.pallas import tpu as pltpu
from jax.experimental.pallas import tpu_sc as plsc

BATCH = 4096
SEQ = 50
SEQP = 56
EMB = 300
EMBP = 384
VOCAB = 100000
NP = BATCH * SEQP

NC = 2
NS = 16
NW = NC * NS
PER_W = NP // NW
CHUNK = 64
NCH = PER_W // CHUNK

_mesh = plsc.VectorSubcoreMesh(
    core_axis_name="c", subcore_axis_name="s", num_cores=NC, num_subcores=NS
)


@functools.partial(
    pl.kernel,
    out_type=jax.ShapeDtypeStruct((NP, EMBP), jnp.float32),
    mesh=_mesh,
    scratch_types=[
        pltpu.VMEM((PER_W,), jnp.int32),
        pltpu.VMEM((CHUNK, EMBP), jnp.float32),
        pltpu.VMEM((CHUNK, EMBP), jnp.float32),
        pltpu.VMEM((CHUNK, EMBP), jnp.float32),
        pltpu.VMEM((CHUNK, EMBP), jnp.float32),
        pltpu.SemaphoreType.DMA,
        pltpu.SemaphoreType.DMA,
        pltpu.SemaphoreType.DMA,
        pltpu.SemaphoreType.DMA,
        pltpu.SemaphoreType.DMA,
        pltpu.SemaphoreType.DMA,
        pltpu.SemaphoreType.DMA,
        pltpu.SemaphoreType.DMA,
    ],
    compiler_params=pltpu.CompilerParams(use_tc_tiling_on_sc=True),
)
def _emb_lookup(idx_hbm, table_hbm, tail_hbm, out_hbm, idx_v,
                buf0, buf1, buf2, buf3,
                gsem0, gsem1, gsem2, gsem3, ssem0, ssem1, ssem2, ssem3):
    wid = lax.axis_index("s") * NC + lax.axis_index("c")
    base = wid * PER_W
    pltpu.sync_copy(idx_hbm.at[pl.ds(base, PER_W)], idx_v)

    bufs = (buf0, buf1, buf2, buf3)
    gsems = (gsem0, gsem1, gsem2, gsem3)
    ssems = (ssem0, ssem1, ssem2, ssem3)

    def start_gathers(g, b):
        idx = idx_v.at[pl.ds(g * CHUNK, CHUNK)]
        pltpu.async_copy(table_hbm.at[idx, pl.ds(0, 128)],
                         bufs[b].at[:, pl.ds(0, 128)], gsems[b])
        pltpu.async_copy(table_hbm.at[idx, pl.ds(128, 128)],
                         bufs[b].at[:, pl.ds(128, 128)], gsems[b])
        pltpu.async_copy(tail_hbm.at[idx],
                         bufs[b].at[:, pl.ds(256, 128)], gsems[b])

    def wait_gathers(b):
        pltpu.make_async_copy(out_hbm.at[pl.ds(0, CHUNK)], bufs[b],
                              gsems[b]).wait()

    def start_store(g, b):
        pltpu.async_copy(bufs[b], out_hbm.at[pl.ds(base + g * CHUNK, CHUNK)],
                         ssems[b])

    def wait_store(b):
        pltpu.make_async_copy(bufs[b], out_hbm.at[pl.ds(0, CHUNK)],
                              ssems[b]).wait()

    for b in range(4):
        start_gathers(b, b)

    @pl.loop(0, NCH, step=4)
    def _quad(g):
        for b in range(4):
            wait_gathers(b)
            start_store(g + b, b)
        for b in range(4):
            @pl.when(g + b + 4 < NCH)
            def _(b=b):
                wait_store(b)
                start_gathers(g + b + 4, b)

    for b in range(4):
        wait_store(b)


def kernel(indices, table):
    dummy = (jnp.arange(BATCH * (SEQP - SEQ), dtype=jnp.int32) % VOCAB).reshape(
        BATCH, SEQP - SEQ
    )
    idx_pad = jnp.concatenate([indices, dummy], axis=1).reshape(NP)
    tail = jnp.concatenate(
        [table[:, 256:], jnp.zeros((VOCAB, EMBP - EMB), jnp.float32)], axis=1
    )
    out = _emb_lookup(idx_pad, table, tail)
    return out.reshape(BATCH, SEQP, EMBP)[:, :SEQ, :EMB]
